# Initial kernel scaffold; baseline (speedup 1.0000x reference)
import jax
import jax.numpy as jnp
from jax.experimental import pallas as pl


def kernel(x, edge_index, W1, b1, g1, be1, W2, b2, g2, be2, W3, b3):
    raise NotImplementedError("write your pallas kernel here")


# speedup vs baseline: 17.5077x; 17.5077x over previous
import functools

import jax
import jax.numpy as jnp
from jax import lax
from jax.experimental import pallas as pl
from jax.experimental.pallas import tpu as pltpu
from jax.experimental.pallas import tpu_sc as plsc

_NC = 2
_NS = 16
_EPS = 1e-5


def _pad_n(N):
    unit = 128 * _NS
    return ((N + unit - 1) // unit) * unit


@functools.lru_cache(maxsize=None)
def _make_agg(N, E, Dh, B):
    e_per = E // _NS
    assert e_per * _NS == E and e_per % B == 0 and B % 8 == 0
    Np = _pad_n(N)
    rows_per_tile = Np // _NS
    zchunk = 128
    assert rows_per_tile % zchunk == 0

    mesh = plsc.VectorSubcoreMesh(core_axis_name="c", subcore_axis_name="s")

    @functools.partial(
        pl.kernel,
        mesh=mesh,
        out_type=jax.ShapeDtypeStruct((_NC, Np, Dh), jnp.float32),
        scratch_types=[
            pltpu.VMEM((B,), jnp.int32),
            pltpu.VMEM((B,), jnp.int32),
            pltpu.VMEM((B, Dh), jnp.float32),
            pltpu.VMEM((zchunk, Dh), jnp.float32),
            pltpu.VMEM_SHARED((Np, Dh), jnp.float32),
            pltpu.SemaphoreType.DMA,
        ],
        compiler_params=pltpu.CompilerParams(use_tc_tiling_on_sc=False),
    )
    def k(hs_a, hs_b, src, dst, zrows, out, src_v, dst_v, rows_v, zb, acc, sem):
        c = lax.axis_index("c")
        s = lax.axis_index("s")
        pltpu.sync_copy(zrows, zb)
        row0 = s * rows_per_tile
        for z in range(rows_per_tile // zchunk):
            pltpu.sync_copy(zb, acc.at[pl.ds(row0 + z * zchunk, zchunk)])
        plsc.subcore_barrier()

        base0 = s * e_per

        def run(hs):
            def body(i, carry):
                base = base0 + i * B
                pltpu.sync_copy(src.at[pl.ds(base, B)], src_v)
                pltpu.sync_copy(dst.at[pl.ds(base, B)], dst_v)
                pltpu.async_copy(hs.at[src_v], rows_v, sem).wait()
                pltpu.sync_copy(rows_v, acc.at[dst_v], add=True)
                return carry

            lax.fori_loop(0, e_per // B, body, 0)

        @pl.when(c == 0)
        def _():
            run(hs_a)

        @pl.when(c == 1)
        def _():
            run(hs_b)

        plsc.subcore_barrier()
        pltpu.sync_copy(acc.at[pl.ds(row0, rows_per_tile)],
                        out.at[c, pl.ds(row0, rows_per_tile)])

    return k


@functools.lru_cache(maxsize=None)
def _make_deg(N, E, B):
    D = 16
    NW = _NC * _NS
    e_per = E // NW
    assert e_per * NW == E and e_per % B == 0 and B % 8 == 0
    Np = _pad_n(N)
    rows_per_tile = Np // _NS

    mesh = plsc.VectorSubcoreMesh(core_axis_name="c", subcore_axis_name="s")

    @functools.partial(
        pl.kernel,
        mesh=mesh,
        out_type=jax.ShapeDtypeStruct((_NC, Np, D), jnp.float32),
        scratch_types=[
            pltpu.VMEM((B,), jnp.int32),
            pltpu.VMEM((B, D), jnp.float32),
            pltpu.VMEM((rows_per_tile, D), jnp.float32),
            pltpu.VMEM_SHARED((Np, D), jnp.float32),
        ],
        compiler_params=pltpu.CompilerParams(use_tc_tiling_on_sc=False),
    )
    def k(dst, ones_rows, zrows, out, dst_v, ones_v, zb, acc):
        c = lax.axis_index("c")
        s = lax.axis_index("s")
        wid = c * _NS + s
        pltpu.sync_copy(ones_rows, ones_v)
        pltpu.sync_copy(zrows, zb)
        row0 = s * rows_per_tile
        pltpu.sync_copy(zb, acc.at[pl.ds(row0, rows_per_tile)])
        plsc.subcore_barrier()

        base0 = wid * e_per

        def body(i, carry):
            base = base0 + i * B
            pltpu.sync_copy(dst.at[pl.ds(base, B)], dst_v)
            pltpu.sync_copy(ones_v, acc.at[dst_v], add=True)
            return carry

        lax.fori_loop(0, e_per // B, body, 0)
        plsc.subcore_barrier()
        pltpu.sync_copy(acc.at[pl.ds(row0, rows_per_tile)],
                        out.at[c, pl.ds(row0, rows_per_tile)])

    return k


def _dinv_from_deg(deg_ref, N):
    d16 = deg_ref[0][:N] + deg_ref[1][:N]
    deg = jnp.sum(d16, axis=1, keepdims=True) * (1.0 / 16.0) + 1.0
    return lax.rsqrt(deg)


def _halves(h, dinv):
    hs = h * dinv
    Dh = h.shape[1] // 2
    return hs[:, :Dh], hs[:, Dh:]


def _tc_first(x, W, deg16):
    N = x.shape[0]
    Dh = W.shape[1] // 2

    def body(x_ref, w_ref, deg_ref, hsa_ref, hsb_ref):
        dinv = _dinv_from_deg(deg_ref, N)
        h = jnp.dot(x_ref[...], w_ref[...], preferred_element_type=jnp.float32)
        hsa_ref[...], hsb_ref[...] = _halves(h, dinv)

    return pl.pallas_call(
        body,
        out_shape=[jax.ShapeDtypeStruct((N, Dh), jnp.float32)] * 2,
    )(x, W, deg16)


def _tc_mid(agg, hsa, hsb, deg16, b, g, be, Wn):
    N = hsa.shape[0]
    Dh = Wn.shape[1] // 2

    def body(agg_ref, hsa_ref, hsb_ref, deg_ref, b_ref, g_ref, be_ref, w_ref,
             outa_ref, outb_ref):
        dinv = _dinv_from_deg(deg_ref, N)
        hs = jnp.concatenate([hsa_ref[...], hsb_ref[...]], axis=1)
        a = jnp.concatenate([agg_ref[0][:N], agg_ref[1][:N]], axis=1)
        y = (a + hs) * dinv + b_ref[...]
        mean = jnp.mean(y, axis=0, keepdims=True)
        var = jnp.mean((y - mean) ** 2, axis=0, keepdims=True)
        z = g_ref[...] * (y - mean) * lax.rsqrt(var + _EPS) + be_ref[...]
        r = jnp.maximum(z, 0.0)
        h = jnp.dot(r, w_ref[...], preferred_element_type=jnp.float32)
        outa_ref[...], outb_ref[...] = _halves(h, dinv)

    return pl.pallas_call(
        body,
        out_shape=[jax.ShapeDtypeStruct((N, Dh), jnp.float32)] * 2,
    )(agg, hsa, hsb, deg16, b.reshape(1, -1), g.reshape(1, -1),
      be.reshape(1, -1), Wn)


def _tc_last(agg, hsa, hsb, deg16, b):
    N = hsa.shape[0]
    D = hsa.shape[1] * 2

    def body(agg_ref, hsa_ref, hsb_ref, deg_ref, b_ref, out_ref):
        dinv = _dinv_from_deg(deg_ref, N)
        hs = jnp.concatenate([hsa_ref[...], hsb_ref[...]], axis=1)
        a = jnp.concatenate([agg_ref[0][:N], agg_ref[1][:N]], axis=1)
        out_ref[...] = (a + hs) * dinv + b_ref[...]

    return pl.pallas_call(
        body,
        out_shape=jax.ShapeDtypeStruct((N, D), jnp.float32),
    )(agg, hsa, hsb, deg16, b.reshape(1, -1))


def kernel(x, edge_index, W1, b1, g1, be1, W2, b2, g2, be2, W3, b3):
    N = x.shape[0]
    E = edge_index.shape[1]
    D_hid = W1.shape[1]
    D_out = W3.shape[1]
    B = 400

    src = edge_index[0].astype(jnp.int32)
    dst = edge_index[1].astype(jnp.int32)

    rows_per_tile = _pad_n(N) // _NS
    zchunk = 128
    z16 = jnp.zeros((rows_per_tile, 16), jnp.float32)
    ones16 = jnp.ones((B, 16), jnp.float32)
    z_hid = jnp.zeros((zchunk, D_hid // 2), jnp.float32)
    z_out = jnp.zeros((zchunk, D_out // 2), jnp.float32)

    deg16 = _make_deg(N, E, B)(dst, ones16, z16)

    agg_hid = _make_agg(N, E, D_hid // 2, B)
    agg_out = _make_agg(N, E, D_out // 2, B)

    hs1a, hs1b = _tc_first(x, W1, deg16)
    a1 = agg_hid(hs1a, hs1b, src, dst, z_hid)
    hs2a, hs2b = _tc_mid(a1, hs1a, hs1b, deg16, b1, g1, be1, W2)
    a2 = agg_hid(hs2a, hs2b, src, dst, z_hid)
    hs3a, hs3b = _tc_mid(a2, hs2a, hs2b, deg16, b2, g2, be2, W3)
    a3 = agg_out(hs3a, hs3b, src, dst, z_out)
    return _tc_last(a3, hs3a, hs3b, deg16, b3)


# speedup vs baseline: 28.4195x; 1.6233x over previous
import functools

import jax
import jax.numpy as jnp
from jax import lax
from jax.experimental import pallas as pl
from jax.experimental.pallas import tpu as pltpu
from jax.experimental.pallas import tpu_sc as plsc

_NC = 2
_NS = 16
_EPS = 1e-5


def _pad_n(N):
    unit = 128 * _NS
    return ((N + unit - 1) // unit) * unit


@functools.lru_cache(maxsize=None)
def _make_agg(N, E, Dh, B):
    e_per = E // _NS
    assert e_per * _NS == E and e_per % B == 0 and B % 8 == 0
    C = e_per // B
    Np = _pad_n(N)
    rows_per_tile = Np // _NS

    mesh = plsc.VectorSubcoreMesh(core_axis_name="c", subcore_axis_name="s")

    @functools.partial(
        pl.kernel,
        mesh=mesh,
        out_type=pltpu.HBM((_NC, Np, Dh), jnp.float32),
        scratch_types=[
            pltpu.VMEM((2, 2, B), jnp.int32),
            pltpu.VMEM((2, B, Dh), jnp.float32),
            pltpu.VMEM_SHARED((Np, Dh), jnp.float32),
            pltpu.SemaphoreType.DMA,
            pltpu.SemaphoreType.DMA,
        ],
        compiler_params=pltpu.CompilerParams(use_tc_tiling_on_sc=False),
    )
    def k(hs, sd3, zrows, out, sd2, rows2, acc, semg, sems):
        c = lax.axis_index("c")
        s = lax.axis_index("s")
        row0 = s * rows_per_tile
        pltpu.sync_copy(zrows, acc.at[pl.ds(row0, rows_per_tile)])

        pltpu.sync_copy(sd3.at[s, 0], sd2.at[0])
        pltpu.async_copy(hs.at[c].at[sd2.at[0, 0]], rows2.at[0], semg)
        plsc.subcore_barrier()

        def drain_gather():
            pltpu.make_async_copy(hs.at[c].at[sd2.at[0, 0]], rows2.at[0],
                                  semg).wait()

        def drain_scatter():
            pltpu.make_async_copy(rows2.at[0], acc.at[sd2.at[0, 1]],
                                  sems).wait()

        def step(i, b):
            @pl.when(i >= 1)
            def _():
                drain_scatter()

            @pl.when(i + 1 < C)
            def _():
                pltpu.sync_copy(sd3.at[s, i + 1], sd2.at[1 - b])
                pltpu.async_copy(hs.at[c].at[sd2.at[1 - b, 0]],
                                 rows2.at[1 - b], semg)

            drain_gather()
            pltpu.async_copy(rows2.at[b], acc.at[sd2.at[b, 1]], sems,
                             add=True)

        def body(i, carry):
            @pl.when(lax.rem(i, 2) == 0)
            def _():
                step(i, 0)

            @pl.when(lax.rem(i, 2) == 1)
            def _():
                step(i, 1)

            return carry

        lax.fori_loop(0, C, body, 0)
        drain_scatter()

        plsc.subcore_barrier()
        pltpu.sync_copy(acc.at[pl.ds(row0, rows_per_tile)],
                        out.at[c, pl.ds(row0, rows_per_tile)])

    return k


@functools.lru_cache(maxsize=None)
def _make_deg(N, E, B):
    D = 16
    NW = _NC * _NS
    e_per = E // NW
    assert e_per * NW == E and e_per % B == 0 and B % 8 == 0
    C = e_per // B
    Np = _pad_n(N)
    rows_per_tile = Np // _NS

    mesh = plsc.VectorSubcoreMesh(core_axis_name="c", subcore_axis_name="s")

    @functools.partial(
        pl.kernel,
        mesh=mesh,
        out_type=pltpu.HBM((_NC, Np, D), jnp.float32),
        scratch_types=[
            pltpu.VMEM((2, B), jnp.int32),
            pltpu.VMEM((B, D), jnp.float32),
            pltpu.VMEM_SHARED((Np, D), jnp.float32),
            pltpu.SemaphoreType.DMA,
        ],
        compiler_params=pltpu.CompilerParams(use_tc_tiling_on_sc=False),
    )
    def k(dstW, ones_rows, zrows, out, dst2, ones_v, acc, sems):
        c = lax.axis_index("c")
        s = lax.axis_index("s")
        wid = c * _NS + s
        row0 = s * rows_per_tile
        pltpu.sync_copy(zrows, acc.at[pl.ds(row0, rows_per_tile)])
        pltpu.sync_copy(ones_rows, ones_v)
        pltpu.sync_copy(dstW.at[wid, 0], dst2.at[0])
        plsc.subcore_barrier()

        def drain_scatter():
            pltpu.make_async_copy(ones_v, acc.at[dst2.at[0]], sems).wait()

        def step(i, b):
            @pl.when(i >= 1)
            def _():
                drain_scatter()

            @pl.when(i + 1 < C)
            def _():
                pltpu.sync_copy(dstW.at[wid, i + 1], dst2.at[1 - b])

            pltpu.async_copy(ones_v, acc.at[dst2.at[b]], sems, add=True)

        def body(i, carry):
            @pl.when(lax.rem(i, 2) == 0)
            def _():
                step(i, 0)

            @pl.when(lax.rem(i, 2) == 1)
            def _():
                step(i, 1)

            return carry

        lax.fori_loop(0, C, body, 0)
        drain_scatter()

        plsc.subcore_barrier()
        pltpu.sync_copy(acc.at[pl.ds(row0, rows_per_tile)],
                        out.at[c, pl.ds(row0, rows_per_tile)])

    return k


def _dinv_from_deg(deg_ref, N):
    d16 = deg_ref[0][:N] + deg_ref[1][:N]
    deg = jnp.sum(d16, axis=1, keepdims=True) * (1.0 / 16.0) + 1.0
    return lax.rsqrt(deg)


def _store_stacked(hs_ref, h, dinv):
    Dh = h.shape[1] // 2
    hs = h * dinv
    hs_ref[0] = hs[:, :Dh]
    hs_ref[1] = hs[:, Dh:]


def _bn_relu_half(a, hs, dinv, b, g, be):
    y = (a + hs) * dinv + b
    mean = jnp.mean(y, axis=0, keepdims=True)
    var = jnp.mean((y - mean) ** 2, axis=0, keepdims=True)
    z = g * (y - mean) * lax.rsqrt(var + _EPS) + be
    return jnp.maximum(z, 0.0)


def _tc_first(x, W, deg16):
    N = x.shape[0]
    Dh = W.shape[1] // 2

    def body(x_ref, w_ref, deg_ref, hs_ref):
        dinv = _dinv_from_deg(deg_ref, N)
        h = jnp.dot(x_ref[...], w_ref[...], preferred_element_type=jnp.float32)
        _store_stacked(hs_ref, h, dinv)

    return pl.pallas_call(
        body,
        out_shape=jax.ShapeDtypeStruct((2, N, Dh), jnp.float32),
    )(x, W, deg16)


def _tc_mid(agg, hs_stk, deg16, b, g, be, Wn):
    N = hs_stk.shape[1]
    Dh = Wn.shape[1] // 2

    def body(agg_ref, hs_stk_ref, deg_ref, b_ref, g_ref, be_ref, w_ref,
             out_ref):
        dinv = _dinv_from_deg(deg_ref, N)
        Dp = hs_stk_ref.shape[2]
        h = None
        for hh in range(2):
            sl = slice(hh * Dp, (hh + 1) * Dp)
            r = _bn_relu_half(agg_ref[hh][:N], hs_stk_ref[hh], dinv,
                              b_ref[:, sl], g_ref[:, sl], be_ref[:, sl])
            p = jnp.dot(r, w_ref[sl, :], preferred_element_type=jnp.float32)
            h = p if h is None else h + p
        _store_stacked(out_ref, h, dinv)

    return pl.pallas_call(
        body,
        out_shape=jax.ShapeDtypeStruct((2, N, Dh), jnp.float32),
    )(agg, hs_stk, deg16, b.reshape(1, -1), g.reshape(1, -1),
      be.reshape(1, -1), Wn)


def _tc_last(agg, hs_stk, deg16, b):
    N = hs_stk.shape[1]
    D = hs_stk.shape[2] * 2

    def body(agg_ref, hs_stk_ref, deg_ref, b_ref, out_ref):
        dinv = _dinv_from_deg(deg_ref, N)
        Dp = hs_stk_ref.shape[2]
        for hh in range(2):
            sl = slice(hh * Dp, (hh + 1) * Dp)
            out_ref[:, sl] = ((agg_ref[hh][:N] + hs_stk_ref[hh]) * dinv
                              + b_ref[:, sl])

    return pl.pallas_call(
        body,
        out_shape=jax.ShapeDtypeStruct((N, D), jnp.float32),
    )(agg, hs_stk, deg16, b.reshape(1, -1))


def kernel(x, edge_index, W1, b1, g1, be1, W2, b2, g2, be2, W3, b3):
    N = x.shape[0]
    E = edge_index.shape[1]
    D_hid = W1.shape[1]
    D_out = W3.shape[1]
    B = 400
    B_out = 800
    B_deg = 1000

    src = edge_index[0].astype(jnp.int32)
    dst = edge_index[1].astype(jnp.int32)
    e_per = E // _NS

    def _sd(Bc):
        return jnp.stack([src.reshape(_NS, e_per // Bc, Bc),
                          dst.reshape(_NS, e_per // Bc, Bc)], axis=2)

    sd3 = _sd(B)
    sd3_out = _sd(B_out)
    NW = _NC * _NS
    dstW = dst.reshape(NW, (E // NW) // B_deg, B_deg)

    rows_per_tile = _pad_n(N) // _NS
    z16 = jnp.zeros((rows_per_tile, 16), jnp.float32)
    ones16 = jnp.ones((B_deg, 16), jnp.float32)
    z_hid = jnp.zeros((rows_per_tile, D_hid // 2), jnp.float32)
    z_out = jnp.zeros((rows_per_tile, D_out // 2), jnp.float32)

    deg16 = _make_deg(N, E, B_deg)(dstW, ones16, z16)

    agg_hid = _make_agg(N, E, D_hid // 2, B)
    agg_out = _make_agg(N, E, D_out // 2, B_out)

    hs1 = _tc_first(x, W1, deg16)
    a1 = agg_hid(hs1, sd3, z_hid)
    hs2 = _tc_mid(a1, hs1, deg16, b1, g1, be1, W2)
    a2 = agg_hid(hs2, sd3, z_hid)
    hs3 = _tc_mid(a2, hs2, deg16, b2, g2, be2, W3)
    a3 = agg_out(hs3, sd3_out, z_out)
    return _tc_last(a3, hs3, deg16, b3)


# speedup vs baseline: 33.4637x; 1.1775x over previous
import functools

import jax
import jax.numpy as jnp
from jax import lax
from jax.experimental import pallas as pl
from jax.experimental.pallas import tpu as pltpu
from jax.experimental.pallas import tpu_sc as plsc

_NC = 2
_NS = 16
_EPS = 1e-5


def _pad_n(N):
    unit = 128 * _NS
    return ((N + unit - 1) // unit) * unit


@functools.lru_cache(maxsize=None)
def _make_agg(N, E, Dh, B):
    e_per = E // _NS
    assert e_per * _NS == E and e_per % B == 0 and B % 8 == 0
    C = e_per // B
    Np = _pad_n(N)
    rows_per_tile = Np // _NS

    mesh = plsc.VectorSubcoreMesh(core_axis_name="c", subcore_axis_name="s")

    @functools.partial(
        pl.kernel,
        mesh=mesh,
        out_type=pltpu.HBM((_NC, Np, Dh), jnp.float32),
        scratch_types=[
            pltpu.VMEM((3, 2, B), jnp.int32),
            pltpu.VMEM((2, B, Dh), jnp.float32),
            pltpu.VMEM_SHARED((Np, Dh), jnp.float32),
            pltpu.SemaphoreType.DMA,
            pltpu.SemaphoreType.DMA,
            pltpu.SemaphoreType.DMA,
        ],
        compiler_params=pltpu.CompilerParams(use_tc_tiling_on_sc=False),
    )
    def k(hs, sd3, zrows, out, sd2, rows2, acc, semg, sems, semi):
        c = lax.axis_index("c")
        s = lax.axis_index("s")
        row0 = s * rows_per_tile
        pltpu.sync_copy(zrows, acc.at[pl.ds(row0, rows_per_tile)])

        pltpu.sync_copy(sd3.at[s, 0], sd2.at[0])
        pltpu.async_copy(hs.at[c].at[sd2.at[0, 0]], rows2.at[0], semg)
        if C > 1:
            pltpu.async_copy(sd3.at[s, 1], sd2.at[1], semi)
        plsc.subcore_barrier()

        def drain_gather():
            pltpu.make_async_copy(hs.at[c].at[sd2.at[0, 0]], rows2.at[0],
                                  semg).wait()

        def drain_scatter():
            pltpu.make_async_copy(rows2.at[0], acc.at[sd2.at[0, 1]],
                                  sems).wait()

        def drain_idx():
            pltpu.make_async_copy(sd3.at[s, 0], sd2.at[0], semi).wait()

        def step(i, b, t):
            @pl.when(i >= 1)
            def _():
                drain_scatter()

            @pl.when(i + 2 < C)
            def _():
                pltpu.async_copy(sd3.at[s, i + 2], sd2.at[(t + 2) % 3], semi)

            @pl.when(i + 1 < C)
            def _():
                drain_idx()
                pltpu.async_copy(hs.at[c].at[sd2.at[(t + 1) % 3, 0]],
                                 rows2.at[1 - b], semg)

            drain_gather()
            pltpu.async_copy(rows2.at[b], acc.at[sd2.at[t, 1]], sems,
                             add=True)

        def body(i, carry):
            for r in range(6):
                @pl.when(lax.rem(i, 6) == r)
                def _(r=r):
                    step(i, r % 2, r % 3)

            return carry

        lax.fori_loop(0, C, body, 0)
        drain_scatter()

        plsc.subcore_barrier()
        pltpu.sync_copy(acc.at[pl.ds(row0, rows_per_tile)],
                        out.at[c, pl.ds(row0, rows_per_tile)])

    return k


@functools.lru_cache(maxsize=None)
def _make_deg(N, E, B):
    D = 16
    NW = _NC * _NS
    e_per = E // NW
    assert e_per * NW == E and e_per % B == 0 and B % 8 == 0
    C = e_per // B
    Np = _pad_n(N)
    rows_per_tile = Np // _NS

    mesh = plsc.VectorSubcoreMesh(core_axis_name="c", subcore_axis_name="s")

    @functools.partial(
        pl.kernel,
        mesh=mesh,
        out_type=pltpu.HBM((_NC, Np, D), jnp.float32),
        scratch_types=[
            pltpu.VMEM((2, B), jnp.int32),
            pltpu.VMEM((B, D), jnp.float32),
            pltpu.VMEM_SHARED((Np, D), jnp.float32),
            pltpu.SemaphoreType.DMA,
        ],
        compiler_params=pltpu.CompilerParams(use_tc_tiling_on_sc=False),
    )
    def k(dstW, ones_rows, zrows, out, dst2, ones_v, acc, sems):
        c = lax.axis_index("c")
        s = lax.axis_index("s")
        wid = c * _NS + s
        row0 = s * rows_per_tile
        pltpu.sync_copy(zrows, acc.at[pl.ds(row0, rows_per_tile)])
        pltpu.sync_copy(ones_rows, ones_v)
        pltpu.sync_copy(dstW.at[wid, 0], dst2.at[0])
        plsc.subcore_barrier()

        def drain_scatter():
            pltpu.make_async_copy(ones_v, acc.at[dst2.at[0]], sems).wait()

        def step(i, b):
            @pl.when(i >= 1)
            def _():
                drain_scatter()

            @pl.when(i + 1 < C)
            def _():
                pltpu.sync_copy(dstW.at[wid, i + 1], dst2.at[1 - b])

            pltpu.async_copy(ones_v, acc.at[dst2.at[b]], sems, add=True)

        def body(i, carry):
            @pl.when(lax.rem(i, 2) == 0)
            def _():
                step(i, 0)

            @pl.when(lax.rem(i, 2) == 1)
            def _():
                step(i, 1)

            return carry

        lax.fori_loop(0, C, body, 0)
        drain_scatter()

        plsc.subcore_barrier()
        pltpu.sync_copy(acc.at[pl.ds(row0, rows_per_tile)],
                        out.at[c, pl.ds(row0, rows_per_tile)])

    return k


def _dinv_from_deg(deg_ref, N):
    d16 = deg_ref[0][:N] + deg_ref[1][:N]
    deg = jnp.sum(d16, axis=1, keepdims=True) * (1.0 / 16.0) + 1.0
    return lax.rsqrt(deg)


def _store_stacked(hs_ref, h, dinv):
    Dh = h.shape[1] // 2
    hs = h * dinv
    hs_ref[0] = hs[:, :Dh]
    hs_ref[1] = hs[:, Dh:]


def _bn_relu_half(a, hs, dinv, b, g, be):
    y = (a + hs) * dinv + b
    mean = jnp.mean(y, axis=0, keepdims=True)
    var = jnp.mean((y - mean) ** 2, axis=0, keepdims=True)
    z = g * (y - mean) * lax.rsqrt(var + _EPS) + be
    return jnp.maximum(z, 0.0)


def _tc_first(x, W, deg16):
    N = x.shape[0]
    Dh = W.shape[1] // 2

    def body(x_ref, w_ref, deg_ref, hs_ref):
        dinv = _dinv_from_deg(deg_ref, N)
        h = jnp.dot(x_ref[...], w_ref[...], preferred_element_type=jnp.float32)
        _store_stacked(hs_ref, h, dinv)

    return pl.pallas_call(
        body,
        out_shape=jax.ShapeDtypeStruct((2, N, Dh), jnp.float32),
    )(x, W, deg16)


def _tc_mid(agg, hs_stk, deg16, b, g, be, Wn):
    N = hs_stk.shape[1]
    Dh = Wn.shape[1] // 2

    def body(agg_ref, hs_stk_ref, deg_ref, b_ref, g_ref, be_ref, w_ref,
             out_ref):
        dinv = _dinv_from_deg(deg_ref, N)
        Dp = hs_stk_ref.shape[2]
        h = None
        for hh in range(2):
            sl = slice(hh * Dp, (hh + 1) * Dp)
            r = _bn_relu_half(agg_ref[hh][:N], hs_stk_ref[hh], dinv,
                              b_ref[:, sl], g_ref[:, sl], be_ref[:, sl])
            p = jnp.dot(r, w_ref[sl, :], preferred_element_type=jnp.float32)
            h = p if h is None else h + p
        _store_stacked(out_ref, h, dinv)

    return pl.pallas_call(
        body,
        out_shape=jax.ShapeDtypeStruct((2, N, Dh), jnp.float32),
    )(agg, hs_stk, deg16, b.reshape(1, -1), g.reshape(1, -1),
      be.reshape(1, -1), Wn)


def _tc_last(agg, hs_stk, deg16, b):
    N = hs_stk.shape[1]
    D = hs_stk.shape[2] * 2

    def body(agg_ref, hs_stk_ref, deg_ref, b_ref, out_ref):
        dinv = _dinv_from_deg(deg_ref, N)
        Dp = hs_stk_ref.shape[2]
        for hh in range(2):
            sl = slice(hh * Dp, (hh + 1) * Dp)
            out_ref[:, sl] = ((agg_ref[hh][:N] + hs_stk_ref[hh]) * dinv
                              + b_ref[:, sl])

    return pl.pallas_call(
        body,
        out_shape=jax.ShapeDtypeStruct((N, D), jnp.float32),
    )(agg, hs_stk, deg16, b.reshape(1, -1))


def kernel(x, edge_index, W1, b1, g1, be1, W2, b2, g2, be2, W3, b3):
    N = x.shape[0]
    E = edge_index.shape[1]
    D_hid = W1.shape[1]
    D_out = W3.shape[1]
    B = 400
    B_out = 800
    B_deg = 1000

    src = edge_index[0].astype(jnp.int32)
    dst = edge_index[1].astype(jnp.int32)
    e_per = E // _NS

    def _sd(Bc):
        return jnp.stack([src.reshape(_NS, e_per // Bc, Bc),
                          dst.reshape(_NS, e_per // Bc, Bc)], axis=2)

    sd3 = _sd(B)
    sd3_out = _sd(B_out)
    NW = _NC * _NS
    dstW = dst.reshape(NW, (E // NW) // B_deg, B_deg)

    rows_per_tile = _pad_n(N) // _NS
    z16 = jnp.zeros((rows_per_tile, 16), jnp.float32)
    ones16 = jnp.ones((B_deg, 16), jnp.float32)
    z_hid = jnp.zeros((rows_per_tile, D_hid // 2), jnp.float32)
    z_out = jnp.zeros((rows_per_tile, D_out // 2), jnp.float32)

    deg16 = _make_deg(N, E, B_deg)(dstW, ones16, z16)

    agg_hid = _make_agg(N, E, D_hid // 2, B)
    agg_out = _make_agg(N, E, D_out // 2, B_out)

    hs1 = _tc_first(x, W1, deg16)
    a1 = agg_hid(hs1, sd3, z_hid)
    hs2 = _tc_mid(a1, hs1, deg16, b1, g1, be1, W2)
    a2 = agg_hid(hs2, sd3, z_hid)
    hs3 = _tc_mid(a2, hs2, deg16, b2, g2, be2, W3)
    a3 = agg_out(hs3, sd3_out, z_out)
    return _tc_last(a3, hs3, deg16, b3)
